# baseline (speedup 1.0000x reference)
import functools
import jax
import jax.numpy as jnp
from jax import lax
from jax.experimental import pallas as pl
from jax.experimental.pallas import tpu as pltpu
from jax.experimental.pallas import tpu_sc as plsc


def _sc_body(x_hbm, out_hbm, in0, in1, dil, buf_zero,
             gsem0, gsem1, dsem, zsem):
    B, H, W, C = x_hbm.shape
    NC, NS = 2, 16
    n_rows = B * H // (NC * NS)
    wid = lax.axis_index("s") * NC + lax.axis_index("c")
    zv = jnp.zeros((16,), jnp.float32)
    nvec = C // 16

    def _zero_dil(r, _):
        for j in range(nvec):
            dil[r, pl.ds(16 * j, 16)] = zv
        return 0

    def _zero_buf(r, _):
        for j in range(nvec):
            buf_zero[r, pl.ds(16 * j, 16)] = zv
        return 0

    lax.fori_loop(0, 2 * W, _zero_dil, 0)
    lax.fori_loop(0, W // 2, _zero_buf, 0)

    def _src(t):
        g = wid * n_rows + t
        return g // H, g % H

    def _gather(t, buf, sem):
        b, h = _src(t)
        return pltpu.async_copy(x_hbm.at[b, h], buf, sem)

    def _wait_gather(t, buf, sem):
        b, h = _src(t)
        pltpu.make_async_copy(x_hbm.at[b, h], buf, sem).wait()

    def _scatter(t, buf, sem, odd):
        b, h = _src(t)
        return pltpu.async_copy(buf, out_hbm.at[b, 2 * h + odd], sem)

    def _wait_scatter(t, buf, sem, odd):
        b, h = _src(t)
        pltpu.make_async_copy(buf, out_hbm.at[b, 2 * h + odd], sem).wait()

    def _scatter_zero(t):
        b, h = _src(t)
        for q in range(4):
            pltpu.async_copy(
                buf_zero,
                out_hbm.at[b, 2 * h + 1, pl.ds(q * (W // 2), W // 2)],
                zsem)

    def _wait_zero(t):
        b, h = _src(t)
        for q in range(4):
            pltpu.make_async_copy(
                buf_zero,
                out_hbm.at[b, 2 * h + 1, pl.ds(q * (W // 2), W // 2)],
                zsem).wait()

    def _dilate(buf_in, buf_dil):
        def body(w, _):
            for j in range(nvec):
                buf_dil[2 * w, pl.ds(16 * j, 16)] = (
                    buf_in[w, pl.ds(16 * j, 16)])
            return 0
        lax.fori_loop(0, W, body, 0)

    _gather(0, in0, gsem0)

    def _iter(i, _):
        tA = 2 * i
        tB = 2 * i + 1

        @pl.when(i > 0)
        def _():
            _wait_scatter(tA - 1, dil, dsem, 0)
        _wait_gather(tA, in0, gsem0)
        _gather(tB, in1, gsem1)
        _dilate(in0, dil)
        _scatter(tA, dil, dsem, 0)

        @pl.when(i > 0)
        def _():
            _wait_zero(tA - 1)
        _scatter_zero(tA)

        _wait_scatter(tA, dil, dsem, 0)
        _wait_gather(tB, in1, gsem1)

        @pl.when(i < (n_rows // 2) - 1)
        def _():
            _gather(tB + 1, in0, gsem0)
        _dilate(in1, dil)
        _scatter(tB, dil, dsem, 0)
        _wait_zero(tA)
        _scatter_zero(tB)
        return 0

    lax.fori_loop(0, n_rows // 2, _iter, 0)

    _wait_scatter(n_rows - 1, dil, dsem, 0)
    _wait_zero(n_rows - 1)


def kernel(inputs):
    B, H, W, C = inputs.shape
    mesh = plsc.VectorSubcoreMesh(core_axis_name="c", subcore_axis_name="s")
    k = functools.partial(
        pl.kernel,
        mesh=mesh,
        out_type=jax.ShapeDtypeStruct((B, 2 * H, 2 * W, C), inputs.dtype),
        scratch_types=[
            pltpu.VMEM((W, C), jnp.float32),
            pltpu.VMEM((W, C), jnp.float32),
            pltpu.VMEM((2 * W, C), jnp.float32),
            pltpu.VMEM((W // 2, C), jnp.float32),
            pltpu.SemaphoreType.DMA,
            pltpu.SemaphoreType.DMA,
            pltpu.SemaphoreType.DMA,
            pltpu.SemaphoreType.DMA,
        ],
    )(_sc_body)
    return k(inputs)
